# baseline (speedup 1.0000x reference)
import functools

import jax
import jax.numpy as jnp
from jax import lax
from jax.experimental import pallas as pl
from jax.experimental.pallas import tpu as pltpu
from jax.experimental.pallas import tpu_sc as plsc

N = 10000
D = 128
H = 128
C = 64
E = 320000

NC = 2
NS = 16
NW = NC * NS
K = 128
NB = 4
STEPS = 80
EPW = K * STEPS
E_PAD = EPW * NW
N_PAD = 10240
RPT = N_PAD // NS
DW = 16
BR = 1024


def _sc_mesh():
    return plsc.VectorSubcoreMesh(core_axis_name="c", subcore_axis_name="s")


def _spmm_sc(g, src, dst, zeros_blk):
    @functools.partial(
        pl.kernel,
        out_type=jax.ShapeDtypeStruct((NC, N_PAD, H), jnp.float32),
        mesh=_sc_mesh(),
        scratch_types=[
            pltpu.VMEM((K,), jnp.int32),
            pltpu.VMEM((K,), jnp.int32),
            pltpu.VMEM((2, K, H), jnp.float32),
            pltpu.VMEM_SHARED((N_PAD, H), jnp.float32),
            pltpu.SemaphoreType.DMA,
            pltpu.SemaphoreType.DMA,
        ],
    )
    def run(g_hbm, src_hbm, dst_hbm, z_hbm, out_hbm,
            src_v, dst_v, rows2, acc, gsem, ssem):
        cid = lax.axis_index("c")
        sid = lax.axis_index("s")
        wid = sid * NC + cid
        wbase = wid * EPW

        pltpu.sync_copy(z_hbm, acc.at[pl.ds(sid * RPT, RPT)])
        plsc.subcore_barrier()

        def body(i, carry):
            u = i % 2

            @pl.when(i >= 2)
            def _():
                pltpu.make_async_copy(rows2.at[u], acc.at[dst_v], ssem).wait()

            @pl.when((i >= 1) & (i <= STEPS))
            def _():
                pltpu.make_async_copy(g_hbm.at[src_v], rows2.at[1 - u],
                                      gsem).wait()
                pltpu.sync_copy(dst_hbm.at[pl.ds(wbase + (i - 1) * K, K)],
                                dst_v)
                pltpu.async_copy(rows2.at[1 - u], acc.at[dst_v], ssem,
                                 add=True)

            @pl.when(i < STEPS)
            def _():
                pltpu.sync_copy(src_hbm.at[pl.ds(wbase + i * K, K)], src_v)
                pltpu.async_copy(g_hbm.at[src_v], rows2.at[u], gsem)

            return carry

        lax.fori_loop(0, STEPS + 2, body, 0)

        plsc.subcore_barrier()
        pltpu.sync_copy(acc.at[pl.ds(sid * RPT, RPT)],
                        out_hbm.at[cid, pl.ds(sid * RPT, RPT)])

    return run(g, src, dst, zeros_blk)


def _deg_sc(dst, zeros_deg, ones_blk):
    @functools.partial(
        pl.kernel,
        out_type=jax.ShapeDtypeStruct((NC, N_PAD, DW), jnp.float32),
        mesh=_sc_mesh(),
        scratch_types=[
            pltpu.VMEM((K,), jnp.int32),
            pltpu.VMEM((K, DW), jnp.float32),
            pltpu.VMEM_SHARED((N_PAD, DW), jnp.float32),
        ],
    )
    def run(dst_hbm, z_hbm, ones_hbm, out_hbm, dst_v, ones_v, acc):
        cid = lax.axis_index("c")
        sid = lax.axis_index("s")
        wid = sid * NC + cid
        wbase = wid * EPW

        pltpu.sync_copy(z_hbm, acc.at[pl.ds(sid * RPT, RPT)])
        pltpu.sync_copy(ones_hbm, ones_v)
        plsc.subcore_barrier()

        def body(i, carry):
            pltpu.sync_copy(dst_hbm.at[pl.ds(wbase + i * K, K)], dst_v)
            pltpu.sync_copy(ones_v, acc.at[dst_v], add=True)
            return carry

        lax.fori_loop(0, STEPS, body, 0)

        plsc.subcore_barrier()
        pltpu.sync_copy(acc.at[pl.ds(sid * RPT, RPT)],
                        out_hbm.at[cid, pl.ds(sid * RPT, RPT)])

    return run(dst, zeros_deg, ones_blk)


def _init_tc(degp, x, W0):
    def body(deg_ref, x_ref, w_ref, dis_ref, g_ref):
        deg = deg_ref[...].sum(axis=0).sum(axis=1, keepdims=True) + 1.0
        dis = lax.rsqrt(deg)
        dis_ref[...] = dis
        g_ref[...] = dis * jnp.dot(x_ref[...], w_ref[...],
                                   preferred_element_type=jnp.float32)

    return pl.pallas_call(
        body,
        grid=(N_PAD // BR,),
        in_specs=[
            pl.BlockSpec((NC, BR, DW), lambda i: (0, i, 0)),
            pl.BlockSpec((BR, D), lambda i: (i, 0)),
            pl.BlockSpec((D, H), lambda i: (0, 0)),
        ],
        out_specs=[
            pl.BlockSpec((BR, 1), lambda i: (i, 0)),
            pl.BlockSpec((BR, H), lambda i: (i, 0)),
        ],
        out_shape=[
            jax.ShapeDtypeStruct((N_PAD, 1), jnp.float32),
            jax.ShapeDtypeStruct((N_PAD, H), jnp.float32),
        ],
    )(degp, x, W0)


def _fuse_tc(P, g_prev, dis, b, W_next):
    def body(p_ref, g_ref, dis_ref, b_ref, w_ref, h_ref, gn_ref):
        dis = dis_ref[...]
        h = jnp.maximum(dis * (p_ref[0] + p_ref[1] + g_ref[...]) + b_ref[...], 0.0)
        h_ref[...] = h
        gn_ref[...] = dis * jnp.dot(h, w_ref[...],
                                    preferred_element_type=jnp.float32)

    return pl.pallas_call(
        body,
        grid=(N_PAD // BR,),
        in_specs=[
            pl.BlockSpec((NC, BR, H), lambda i: (0, i, 0)),
            pl.BlockSpec((BR, H), lambda i: (i, 0)),
            pl.BlockSpec((BR, 1), lambda i: (i, 0)),
            pl.BlockSpec((1, H), lambda i: (0, 0)),
            pl.BlockSpec((H, H), lambda i: (0, 0)),
        ],
        out_specs=[
            pl.BlockSpec((BR, H), lambda i: (i, 0)),
            pl.BlockSpec((BR, H), lambda i: (i, 0)),
        ],
        out_shape=[
            jax.ShapeDtypeStruct((N_PAD, H), jnp.float32),
            jax.ShapeDtypeStruct((N_PAD, H), jnp.float32),
        ],
    )(P, g_prev, dis, b, W_next)


def _last_tc(P, g_prev, dis, b, h1, h2, h3, Wp, bp):
    def body(p_ref, g_ref, dis_ref, b_ref, h1_ref, h2_ref, h3_ref,
             wp_ref, bp_ref, o_ref):
        h4 = jnp.maximum(
            dis_ref[...] * (p_ref[0] + p_ref[1] + g_ref[...]) + b_ref[...], 0.0)
        hm = jnp.maximum(jnp.maximum(h1_ref[...], h2_ref[...]),
                         jnp.maximum(h3_ref[...], h4))
        o_ref[...] = jnp.dot(hm, wp_ref[...],
                             preferred_element_type=jnp.float32) + bp_ref[...]

    return pl.pallas_call(
        body,
        grid=(N_PAD // BR,),
        in_specs=[
            pl.BlockSpec((NC, BR, H), lambda i: (0, i, 0)),
            pl.BlockSpec((BR, H), lambda i: (i, 0)),
            pl.BlockSpec((BR, 1), lambda i: (i, 0)),
            pl.BlockSpec((1, H), lambda i: (0, 0)),
            pl.BlockSpec((BR, H), lambda i: (i, 0)),
            pl.BlockSpec((BR, H), lambda i: (i, 0)),
            pl.BlockSpec((BR, H), lambda i: (i, 0)),
            pl.BlockSpec((H, H), lambda i: (0, 0)),
            pl.BlockSpec((1, H), lambda i: (0, 0)),
        ],
        out_specs=pl.BlockSpec((BR, H), lambda i: (i, 0)),
        out_shape=jax.ShapeDtypeStruct((N_PAD, H), jnp.float32),
    )(P, g_prev, dis, b, h1, h2, h3, Wp, bp)


def kernel(x, edge_index, W0, b0, W1, b1, W2, b2, W3, b3, Wp, bp):
    src = edge_index[0].astype(jnp.int32)
    dst = edge_index[1].astype(jnp.int32)
    pad = jnp.full((E_PAD - E,), N, dtype=jnp.int32)
    src_p = jnp.concatenate([src, pad])
    dst_p = jnp.concatenate([dst, pad])
    x_p = jnp.pad(x, ((0, N_PAD - N), (0, 0)))

    zeros_blk = jnp.zeros((RPT, H), jnp.float32)
    zeros_deg = jnp.zeros((RPT, DW), jnp.float32)
    ones_blk = jnp.ones((K, DW), jnp.float32)

    Wp_p = jnp.pad(Wp, ((0, 0), (0, H - C)))
    bp_p = jnp.pad(bp, (0, H - C)).reshape(1, H)

    degp = _deg_sc(dst_p, zeros_deg, ones_blk)
    dis, g = _init_tc(degp, x_p, W0)

    hs = []
    for (b_cur, W_next) in ((b0, W1), (b1, W2), (b2, W3)):
        P = _spmm_sc(g, src_p, dst_p, zeros_blk)
        h, g = _fuse_tc(P, g, dis, b_cur.reshape(1, H), W_next)
        hs.append(h)

    P = _spmm_sc(g, src_p, dst_p, zeros_blk)
    out = _last_tc(P, g, dis, b3.reshape(1, H), hs[0], hs[1], hs[2],
                   Wp_p, bp_p)
    return out[:N, :C]


# speedup vs baseline: 2.1028x; 2.1028x over previous
import functools

import jax
import jax.numpy as jnp
from jax import lax
from jax.experimental import pallas as pl
from jax.experimental.pallas import tpu as pltpu
from jax.experimental.pallas import tpu_sc as plsc

N = 10000
D = 128
H = 128
C = 64
E = 320000

NC = 2
NS = 16
NW = NC * NS
K = 128
NB = 4
STEPS = 80
EPW = K * STEPS
E_PAD = EPW * NW
N_PAD = 10240
RPT = N_PAD // NS
DW = 16
BR = 1024


def _sc_mesh():
    return plsc.VectorSubcoreMesh(core_axis_name="c", subcore_axis_name="s")


def _spmm_sc(g, src, dst, zeros_blk):
    @functools.partial(
        pl.kernel,
        out_type=jax.ShapeDtypeStruct((NC, N_PAD, H), jnp.float32),
        mesh=_sc_mesh(),
        scratch_types=[
            pltpu.VMEM((K,), jnp.int32),
            pltpu.VMEM((K,), jnp.int32),
            pltpu.VMEM((K, H), jnp.float32),
            pltpu.VMEM_SHARED((N_PAD, H), jnp.float32),
            pltpu.SemaphoreType.DMA,
        ],
    )
    def run(g_hbm, src_hbm, dst_hbm, z_hbm, out_hbm,
            src_v, dst_v, rows_v, acc, sem):
        cid = lax.axis_index("c")
        sid = lax.axis_index("s")
        wid = sid * NC + cid
        wbase = wid * EPW

        pltpu.sync_copy(z_hbm, acc.at[pl.ds(sid * RPT, RPT)])
        plsc.subcore_barrier()

        def body(i, carry):
            base = wbase + i * K
            pltpu.sync_copy(src_hbm.at[pl.ds(base, K)], src_v)
            pltpu.sync_copy(dst_hbm.at[pl.ds(base, K)], dst_v)
            pltpu.async_copy(g_hbm.at[src_v], rows_v, sem).wait()
            pltpu.sync_copy(rows_v, acc.at[dst_v], add=True)
            return carry

        lax.fori_loop(0, STEPS, body, 0)

        plsc.subcore_barrier()
        pltpu.sync_copy(acc.at[pl.ds(sid * RPT, RPT)],
                        out_hbm.at[cid, pl.ds(sid * RPT, RPT)])

    return run(g, src, dst, zeros_blk)


def _deg_sc(dst, zeros_deg, ones_blk):
    @functools.partial(
        pl.kernel,
        out_type=jax.ShapeDtypeStruct((NC, N_PAD, DW), jnp.float32),
        mesh=_sc_mesh(),
        scratch_types=[
            pltpu.VMEM((K,), jnp.int32),
            pltpu.VMEM((K, DW), jnp.float32),
            pltpu.VMEM_SHARED((N_PAD, DW), jnp.float32),
        ],
    )
    def run(dst_hbm, z_hbm, ones_hbm, out_hbm, dst_v, ones_v, acc):
        cid = lax.axis_index("c")
        sid = lax.axis_index("s")
        wid = sid * NC + cid
        wbase = wid * EPW

        pltpu.sync_copy(z_hbm, acc.at[pl.ds(sid * RPT, RPT)])
        pltpu.sync_copy(ones_hbm, ones_v)
        plsc.subcore_barrier()

        def body(i, carry):
            pltpu.sync_copy(dst_hbm.at[pl.ds(wbase + i * K, K)], dst_v)
            pltpu.sync_copy(ones_v, acc.at[dst_v], add=True)
            return carry

        lax.fori_loop(0, STEPS, body, 0)

        plsc.subcore_barrier()
        pltpu.sync_copy(acc.at[pl.ds(sid * RPT, RPT)],
                        out_hbm.at[cid, pl.ds(sid * RPT, RPT)])

    return run(dst, zeros_deg, ones_blk)


def _init_tc(degp, x, W0):
    def body(deg_ref, x_ref, w_ref, dis_ref, g_ref):
        deg = deg_ref[...].sum(axis=0).sum(axis=1, keepdims=True) + 1.0
        dis = lax.rsqrt(deg)
        dis_ref[...] = dis
        g_ref[...] = dis * jnp.dot(x_ref[...], w_ref[...],
                                   preferred_element_type=jnp.float32)

    return pl.pallas_call(
        body,
        grid=(N_PAD // BR,),
        in_specs=[
            pl.BlockSpec((NC, BR, DW), lambda i: (0, i, 0)),
            pl.BlockSpec((BR, D), lambda i: (i, 0)),
            pl.BlockSpec((D, H), lambda i: (0, 0)),
        ],
        out_specs=[
            pl.BlockSpec((BR, 1), lambda i: (i, 0)),
            pl.BlockSpec((BR, H), lambda i: (i, 0)),
        ],
        out_shape=[
            jax.ShapeDtypeStruct((N_PAD, 1), jnp.float32),
            jax.ShapeDtypeStruct((N_PAD, H), jnp.float32),
        ],
    )(degp, x, W0)


def _fuse_tc(P, g_prev, dis, b, W_next):
    def body(p_ref, g_ref, dis_ref, b_ref, w_ref, h_ref, gn_ref):
        dis = dis_ref[...]
        h = jnp.maximum(dis * (p_ref[0] + p_ref[1] + g_ref[...]) + b_ref[...], 0.0)
        h_ref[...] = h
        gn_ref[...] = dis * jnp.dot(h, w_ref[...],
                                    preferred_element_type=jnp.float32)

    return pl.pallas_call(
        body,
        grid=(N_PAD // BR,),
        in_specs=[
            pl.BlockSpec((NC, BR, H), lambda i: (0, i, 0)),
            pl.BlockSpec((BR, H), lambda i: (i, 0)),
            pl.BlockSpec((BR, 1), lambda i: (i, 0)),
            pl.BlockSpec((1, H), lambda i: (0, 0)),
            pl.BlockSpec((H, H), lambda i: (0, 0)),
        ],
        out_specs=[
            pl.BlockSpec((BR, H), lambda i: (i, 0)),
            pl.BlockSpec((BR, H), lambda i: (i, 0)),
        ],
        out_shape=[
            jax.ShapeDtypeStruct((N_PAD, H), jnp.float32),
            jax.ShapeDtypeStruct((N_PAD, H), jnp.float32),
        ],
    )(P, g_prev, dis, b, W_next)


def _last_tc(P, g_prev, dis, b, h1, h2, h3, Wp, bp):
    def body(p_ref, g_ref, dis_ref, b_ref, h1_ref, h2_ref, h3_ref,
             wp_ref, bp_ref, o_ref):
        h4 = jnp.maximum(
            dis_ref[...] * (p_ref[0] + p_ref[1] + g_ref[...]) + b_ref[...], 0.0)
        hm = jnp.maximum(jnp.maximum(h1_ref[...], h2_ref[...]),
                         jnp.maximum(h3_ref[...], h4))
        o_ref[...] = jnp.dot(hm, wp_ref[...],
                             preferred_element_type=jnp.float32) + bp_ref[...]

    return pl.pallas_call(
        body,
        grid=(N_PAD // BR,),
        in_specs=[
            pl.BlockSpec((NC, BR, H), lambda i: (0, i, 0)),
            pl.BlockSpec((BR, H), lambda i: (i, 0)),
            pl.BlockSpec((BR, 1), lambda i: (i, 0)),
            pl.BlockSpec((1, H), lambda i: (0, 0)),
            pl.BlockSpec((BR, H), lambda i: (i, 0)),
            pl.BlockSpec((BR, H), lambda i: (i, 0)),
            pl.BlockSpec((BR, H), lambda i: (i, 0)),
            pl.BlockSpec((H, H), lambda i: (0, 0)),
            pl.BlockSpec((1, H), lambda i: (0, 0)),
        ],
        out_specs=pl.BlockSpec((BR, H), lambda i: (i, 0)),
        out_shape=jax.ShapeDtypeStruct((N_PAD, H), jnp.float32),
    )(P, g_prev, dis, b, h1, h2, h3, Wp, bp)


def kernel(x, edge_index, W0, b0, W1, b1, W2, b2, W3, b3, Wp, bp):
    src = edge_index[0].astype(jnp.int32)
    dst = edge_index[1].astype(jnp.int32)
    pad = N + (jnp.arange(E_PAD - E, dtype=jnp.int32) % (N_PAD - N))
    src_p = jnp.concatenate([src, pad])
    dst_p = jnp.concatenate([dst, pad])
    x_p = jnp.pad(x, ((0, N_PAD - N), (0, 0)))

    zeros_blk = jnp.zeros((RPT, H), jnp.float32)
    zeros_deg = jnp.zeros((RPT, DW), jnp.float32)
    ones_blk = jnp.ones((K, DW), jnp.float32)

    Wp_p = jnp.pad(Wp, ((0, 0), (0, H - C)))
    bp_p = jnp.pad(bp, (0, H - C)).reshape(1, H)

    degp = _deg_sc(dst_p, zeros_deg, ones_blk)
    dis, g = _init_tc(degp, x_p, W0)

    hs = []
    for (b_cur, W_next) in ((b0, W1), (b1, W2), (b2, W3)):
        P = _spmm_sc(g, src_p, dst_p, zeros_blk)
        h, g = _fuse_tc(P, g, dis, b_cur.reshape(1, H), W_next)
        hs.append(h)

    P = _spmm_sc(g, src_p, dst_p, zeros_blk)
    out = _last_tc(P, g, dis, b3.reshape(1, H), hs[0], hs[1], hs[2],
                   Wp_p, bp_p)
    return out[:N, :C]


# speedup vs baseline: 2.1543x; 1.0245x over previous
import functools

import jax
import jax.numpy as jnp
from jax import lax
from jax.experimental import pallas as pl
from jax.experimental.pallas import tpu as pltpu
from jax.experimental.pallas import tpu_sc as plsc

N = 10000
D = 128
H = 128
C = 64
E = 320000

NC = 2
NS = 16
NW = NC * NS
K = 128
STEPS = 79
EPW = K * STEPS
E_PAD = EPW * NW
N_PAD = 10240
RPT = N_PAD // NS
DW = 16
BR = 1024


def _sc_mesh():
    return plsc.VectorSubcoreMesh(core_axis_name="c", subcore_axis_name="s")


def _spmm_sc(g, src, dst, zeros_blk):
    @functools.partial(
        pl.kernel,
        out_type=jax.ShapeDtypeStruct((NC, N_PAD, H), jnp.float32),
        mesh=_sc_mesh(),
        scratch_types=[
            pltpu.VMEM((K,), jnp.int32),
            pltpu.VMEM((K,), jnp.int32),
            pltpu.VMEM((K, H), jnp.float32),
            pltpu.VMEM_SHARED((N_PAD, H), jnp.float32),
            pltpu.SemaphoreType.DMA,
        ],
    )
    def run(g_hbm, src_hbm, dst_hbm, z_hbm, out_hbm,
            src_v, dst_v, rows_v, acc, sem):
        cid = lax.axis_index("c")
        sid = lax.axis_index("s")
        wid = sid * NC + cid
        wbase = wid * EPW

        pltpu.sync_copy(z_hbm, acc.at[pl.ds(sid * RPT, RPT)])
        plsc.subcore_barrier()

        def body(i, carry):
            base = wbase + i * K
            pltpu.sync_copy(src_hbm.at[pl.ds(base, K)], src_v)
            pltpu.sync_copy(dst_hbm.at[pl.ds(base, K)], dst_v)
            pltpu.async_copy(g_hbm.at[src_v], rows_v, sem).wait()
            pltpu.sync_copy(rows_v, acc.at[dst_v], add=True)
            return carry

        lax.fori_loop(0, STEPS, body, 0)

        plsc.subcore_barrier()
        pltpu.sync_copy(acc.at[pl.ds(sid * RPT, RPT)],
                        out_hbm.at[cid, pl.ds(sid * RPT, RPT)])

    return run(g, src, dst, zeros_blk)


def _deg_sc(dst, zeros_n):
    @functools.partial(
        pl.kernel,
        out_type=jax.ShapeDtypeStruct((NC, NS, N_PAD), jnp.float32),
        mesh=_sc_mesh(),
        compiler_params=pltpu.CompilerParams(needs_layout_passes=False),
        scratch_types=[
            pltpu.VMEM((K,), jnp.int32),
            pltpu.VMEM((N_PAD,), jnp.float32),
        ],
    )
    def run(dst_hbm, zn_hbm, out_hbm, dst_v, deg_ref):
        cid = lax.axis_index("c")
        sid = lax.axis_index("s")
        wid = sid * NC + cid
        wbase = wid * EPW
        pltpu.sync_copy(zn_hbm, deg_ref)
        ones16 = jnp.full((16,), 1.0, jnp.float32)

        def body(i, carry):
            pltpu.sync_copy(dst_hbm.at[pl.ds(wbase + i * K, K)], dst_v)
            for j in range(K // 16):
                idxv = dst_v[pl.ds(j * 16, 16)]
                plsc.addupdate_scatter(deg_ref, [idxv], ones16)
            return carry

        lax.fori_loop(0, STEPS, body, 0)
        pltpu.sync_copy(deg_ref, out_hbm.at[cid, sid])

    return run(dst, zeros_n)


def _init_tc(degp, x, W0):
    def body(deg_ref, x_ref, w_ref, dis_ref, g_ref):
        deg = deg_ref[...].sum(axis=(0, 1))[:, None] + 1.0
        dis = lax.rsqrt(deg)
        dis_ref[...] = dis
        g_ref[...] = dis * jnp.dot(x_ref[...], w_ref[...],
                                   preferred_element_type=jnp.float32)

    return pl.pallas_call(
        body,
        grid=(N_PAD // BR,),
        in_specs=[
            pl.BlockSpec((NC, NS, BR), lambda i: (0, 0, i)),
            pl.BlockSpec((BR, D), lambda i: (i, 0)),
            pl.BlockSpec((D, H), lambda i: (0, 0)),
        ],
        out_specs=[
            pl.BlockSpec((BR, 1), lambda i: (i, 0)),
            pl.BlockSpec((BR, H), lambda i: (i, 0)),
        ],
        out_shape=[
            jax.ShapeDtypeStruct((N_PAD, 1), jnp.float32),
            jax.ShapeDtypeStruct((N_PAD, H), jnp.float32),
        ],
    )(degp, x, W0)


def _fuse_tc(P, g_prev, dis, b, W_next):
    def body(p_ref, g_ref, dis_ref, b_ref, w_ref, h_ref, gn_ref):
        dis = dis_ref[...]
        h = jnp.maximum(dis * (p_ref[0] + p_ref[1] + g_ref[...]) + b_ref[...], 0.0)
        h_ref[...] = h
        gn_ref[...] = dis * jnp.dot(h, w_ref[...],
                                    preferred_element_type=jnp.float32)

    return pl.pallas_call(
        body,
        grid=(N_PAD // BR,),
        in_specs=[
            pl.BlockSpec((NC, BR, H), lambda i: (0, i, 0)),
            pl.BlockSpec((BR, H), lambda i: (i, 0)),
            pl.BlockSpec((BR, 1), lambda i: (i, 0)),
            pl.BlockSpec((1, H), lambda i: (0, 0)),
            pl.BlockSpec((H, H), lambda i: (0, 0)),
        ],
        out_specs=[
            pl.BlockSpec((BR, H), lambda i: (i, 0)),
            pl.BlockSpec((BR, H), lambda i: (i, 0)),
        ],
        out_shape=[
            jax.ShapeDtypeStruct((N_PAD, H), jnp.float32),
            jax.ShapeDtypeStruct((N_PAD, H), jnp.float32),
        ],
    )(P, g_prev, dis, b, W_next)


def _last_tc(P, g_prev, dis, b, h1, h2, h3, Wp, bp):
    def body(p_ref, g_ref, dis_ref, b_ref, h1_ref, h2_ref, h3_ref,
             wp_ref, bp_ref, o_ref):
        h4 = jnp.maximum(
            dis_ref[...] * (p_ref[0] + p_ref[1] + g_ref[...]) + b_ref[...], 0.0)
        hm = jnp.maximum(jnp.maximum(h1_ref[...], h2_ref[...]),
                         jnp.maximum(h3_ref[...], h4))
        o_ref[...] = jnp.dot(hm, wp_ref[...],
                             preferred_element_type=jnp.float32) + bp_ref[...]

    return pl.pallas_call(
        body,
        grid=(N_PAD // BR,),
        in_specs=[
            pl.BlockSpec((NC, BR, H), lambda i: (0, i, 0)),
            pl.BlockSpec((BR, H), lambda i: (i, 0)),
            pl.BlockSpec((BR, 1), lambda i: (i, 0)),
            pl.BlockSpec((1, H), lambda i: (0, 0)),
            pl.BlockSpec((BR, H), lambda i: (i, 0)),
            pl.BlockSpec((BR, H), lambda i: (i, 0)),
            pl.BlockSpec((BR, H), lambda i: (i, 0)),
            pl.BlockSpec((H, H), lambda i: (0, 0)),
            pl.BlockSpec((1, H), lambda i: (0, 0)),
        ],
        out_specs=pl.BlockSpec((BR, H), lambda i: (i, 0)),
        out_shape=jax.ShapeDtypeStruct((N_PAD, H), jnp.float32),
    )(P, g_prev, dis, b, h1, h2, h3, Wp, bp)


def kernel(x, edge_index, W0, b0, W1, b1, W2, b2, W3, b3, Wp, bp):
    src = edge_index[0].astype(jnp.int32)
    dst = edge_index[1].astype(jnp.int32)
    pad = N + (jnp.arange(E_PAD - E, dtype=jnp.int32) % (N_PAD - N))
    src_p = jnp.concatenate([src, pad])
    dst_p = jnp.concatenate([dst, pad])
    x_p = jnp.pad(x, ((0, N_PAD - N), (0, 0)))

    zeros_blk = jnp.zeros((RPT, H), jnp.float32)
    zeros_n = jnp.zeros((N_PAD,), jnp.float32)

    Wp_p = jnp.pad(Wp, ((0, 0), (0, H - C)))
    bp_p = jnp.pad(bp, (0, H - C)).reshape(1, H)

    degp = _deg_sc(dst_p, zeros_n)
    dis, g = _init_tc(degp, x_p, W0)

    hs = []
    for (b_cur, W_next) in ((b0, W1), (b1, W2), (b2, W3)):
        P = _spmm_sc(g, src_p, dst_p, zeros_blk)
        h, g = _fuse_tc(P, g, dis, b_cur.reshape(1, H), W_next)
        hs.append(h)

    P = _spmm_sc(g, src_p, dst_p, zeros_blk)
    out = _last_tc(P, g, dis, b3.reshape(1, H), hs[0], hs[1], hs[2],
                   Wp_p, bp_p)
    return out[:N, :C]


# speedup vs baseline: 2.8358x; 1.3163x over previous
import functools

import jax
import jax.numpy as jnp
from jax import lax
from jax.experimental import pallas as pl
from jax.experimental.pallas import tpu as pltpu
from jax.experimental.pallas import tpu_sc as plsc

N = 10000
D = 128
H = 128
C = 64
E = 320000

NC = 2
NS = 16
NW = NC * NS
K = 128
STEPS = 80
EPW = K * STEPS
E_PAD = EPW * NW
N_PAD = 10240
RPT = N_PAD // NS
DW = 16
BR = 1024


def _sc_mesh():
    return plsc.VectorSubcoreMesh(core_axis_name="c", subcore_axis_name="s")


def _spmm_sc(g, src, dst, zeros_blk):
    @functools.partial(
        pl.kernel,
        out_type=jax.ShapeDtypeStruct((NC, N_PAD, H), jnp.float32),
        mesh=_sc_mesh(),
        scratch_types=[
            pltpu.VMEM((STEPS, K), jnp.int32),
            pltpu.VMEM((STEPS, K), jnp.int32),
            pltpu.VMEM((K, H), jnp.float32),
            pltpu.VMEM_SHARED((N_PAD, H), jnp.float32),
            pltpu.SemaphoreType.DMA,
        ],
    )
    def run(g_hbm, src_hbm, dst_hbm, z_hbm, out_hbm,
            src_all, dst_all, rows_v, acc, sem):
        cid = lax.axis_index("c")
        sid = lax.axis_index("s")
        wid = sid * NC + cid

        pltpu.sync_copy(z_hbm, acc.at[pl.ds(sid * RPT, RPT)])
        pltpu.sync_copy(src_hbm.at[pl.ds(wid * STEPS, STEPS)], src_all)
        pltpu.sync_copy(dst_hbm.at[pl.ds(wid * STEPS, STEPS)], dst_all)
        plsc.subcore_barrier()

        def body(i, carry):
            pltpu.async_copy(g_hbm.at[src_all.at[i]], rows_v, sem).wait()
            pltpu.sync_copy(rows_v, acc.at[dst_all.at[i]], add=True)
            return carry

        lax.fori_loop(0, STEPS, body, 0)

        plsc.subcore_barrier()
        pltpu.sync_copy(acc.at[pl.ds(sid * RPT, RPT)],
                        out_hbm.at[cid, pl.ds(sid * RPT, RPT)])

    return run(g, src, dst, zeros_blk)


def _deg_sc(dst, zeros_n):
    @functools.partial(
        pl.kernel,
        out_type=jax.ShapeDtypeStruct((NC, NS, N_PAD), jnp.float32),
        mesh=_sc_mesh(),
        compiler_params=pltpu.CompilerParams(needs_layout_passes=False),
        scratch_types=[
            pltpu.VMEM((K,), jnp.int32),
            pltpu.VMEM((N_PAD,), jnp.float32),
        ],
    )
    def run(dst_hbm, zn_hbm, out_hbm, dst_v, deg_ref):
        cid = lax.axis_index("c")
        sid = lax.axis_index("s")
        wid = sid * NC + cid
        wbase = wid * EPW
        pltpu.sync_copy(zn_hbm, deg_ref)
        ones16 = jnp.full((16,), 1.0, jnp.float32)

        def body(i, carry):
            pltpu.sync_copy(dst_hbm.at[pl.ds(wbase + i * K, K)], dst_v)
            for j in range(K // 16):
                idxv = dst_v[pl.ds(j * 16, 16)]
                plsc.addupdate_scatter(deg_ref, [idxv], ones16)
            return carry

        lax.fori_loop(0, STEPS, body, 0)
        pltpu.sync_copy(deg_ref, out_hbm.at[cid, sid])

    return run(dst, zeros_n)


def _init_tc(degp, x, W0):
    def body(deg_ref, x_ref, w_ref, dis_ref, g_ref):
        deg = deg_ref[...].sum(axis=(0, 1))[:, None] + 1.0
        dis = lax.rsqrt(deg)
        dis_ref[...] = dis
        g_ref[...] = dis * jnp.dot(x_ref[...], w_ref[...],
                                   preferred_element_type=jnp.float32)

    return pl.pallas_call(
        body,
        grid=(N_PAD // BR,),
        in_specs=[
            pl.BlockSpec((NC, NS, BR), lambda i: (0, 0, i)),
            pl.BlockSpec((BR, D), lambda i: (i, 0)),
            pl.BlockSpec((D, H), lambda i: (0, 0)),
        ],
        out_specs=[
            pl.BlockSpec((BR, 1), lambda i: (i, 0)),
            pl.BlockSpec((BR, H), lambda i: (i, 0)),
        ],
        out_shape=[
            jax.ShapeDtypeStruct((N_PAD, 1), jnp.float32),
            jax.ShapeDtypeStruct((N_PAD, H), jnp.float32),
        ],
    )(degp, x, W0)


def _fuse_tc(P, g_prev, dis, b, W_next):
    def body(p_ref, g_ref, dis_ref, b_ref, w_ref, h_ref, gn_ref):
        dis = dis_ref[...]
        h = jnp.maximum(dis * (p_ref[0] + p_ref[1] + g_ref[...]) + b_ref[...], 0.0)
        h_ref[...] = h
        gn_ref[...] = dis * jnp.dot(h, w_ref[...],
                                    preferred_element_type=jnp.float32)

    return pl.pallas_call(
        body,
        grid=(N_PAD // BR,),
        in_specs=[
            pl.BlockSpec((NC, BR, H), lambda i: (0, i, 0)),
            pl.BlockSpec((BR, H), lambda i: (i, 0)),
            pl.BlockSpec((BR, 1), lambda i: (i, 0)),
            pl.BlockSpec((1, H), lambda i: (0, 0)),
            pl.BlockSpec((H, H), lambda i: (0, 0)),
        ],
        out_specs=[
            pl.BlockSpec((BR, H), lambda i: (i, 0)),
            pl.BlockSpec((BR, H), lambda i: (i, 0)),
        ],
        out_shape=[
            jax.ShapeDtypeStruct((N_PAD, H), jnp.float32),
            jax.ShapeDtypeStruct((N_PAD, H), jnp.float32),
        ],
    )(P, g_prev, dis, b, W_next)


def _last_tc(P, g_prev, dis, b, h1, h2, h3, Wp, bp):
    def body(p_ref, g_ref, dis_ref, b_ref, h1_ref, h2_ref, h3_ref,
             wp_ref, bp_ref, o_ref):
        h4 = jnp.maximum(
            dis_ref[...] * (p_ref[0] + p_ref[1] + g_ref[...]) + b_ref[...], 0.0)
        hm = jnp.maximum(jnp.maximum(h1_ref[...], h2_ref[...]),
                         jnp.maximum(h3_ref[...], h4))
        o_ref[...] = jnp.dot(hm, wp_ref[...],
                             preferred_element_type=jnp.float32) + bp_ref[...]

    return pl.pallas_call(
        body,
        grid=(N_PAD // BR,),
        in_specs=[
            pl.BlockSpec((NC, BR, H), lambda i: (0, i, 0)),
            pl.BlockSpec((BR, H), lambda i: (i, 0)),
            pl.BlockSpec((BR, 1), lambda i: (i, 0)),
            pl.BlockSpec((1, H), lambda i: (0, 0)),
            pl.BlockSpec((BR, H), lambda i: (i, 0)),
            pl.BlockSpec((BR, H), lambda i: (i, 0)),
            pl.BlockSpec((BR, H), lambda i: (i, 0)),
            pl.BlockSpec((H, H), lambda i: (0, 0)),
            pl.BlockSpec((1, H), lambda i: (0, 0)),
        ],
        out_specs=pl.BlockSpec((BR, H), lambda i: (i, 0)),
        out_shape=jax.ShapeDtypeStruct((N_PAD, H), jnp.float32),
    )(P, g_prev, dis, b, h1, h2, h3, Wp, bp)


def kernel(x, edge_index, W0, b0, W1, b1, W2, b2, W3, b3, Wp, bp):
    src = edge_index[0].astype(jnp.int32)
    dst = edge_index[1].astype(jnp.int32)
    pad = N + (jnp.arange(E_PAD - E, dtype=jnp.int32) % (N_PAD - N))
    src_p = jnp.concatenate([src, pad])
    dst_p = jnp.concatenate([dst, pad])
    src_w = src_p.reshape(NW * STEPS, K)
    dst_w = dst_p.reshape(NW * STEPS, K)
    x_p = jnp.pad(x, ((0, N_PAD - N), (0, 0)))

    zeros_blk = jnp.zeros((RPT, H), jnp.float32)
    zeros_n = jnp.zeros((N_PAD,), jnp.float32)

    Wp_p = jnp.pad(Wp, ((0, 0), (0, H - C)))
    bp_p = jnp.pad(bp, (0, H - C)).reshape(1, H)

    degp = _deg_sc(dst_p, zeros_n)
    dis, g = _init_tc(degp, x_p, W0)

    hs = []
    for (b_cur, W_next) in ((b0, W1), (b1, W2), (b2, W3)):
        P = _spmm_sc(g, src_w, dst_w, zeros_blk)
        h, g = _fuse_tc(P, g, dis, b_cur.reshape(1, H), W_next)
        hs.append(h)

    P = _spmm_sc(g, src_w, dst_w, zeros_blk)
    out = _last_tc(P, g, dis, b3.reshape(1, H), hs[0], hs[1], hs[2],
                   Wp_p, bp_p)
    return out[:N, :C]
